# Initial kernel scaffold; baseline (speedup 1.0000x reference)
import jax
import jax.numpy as jnp
from jax.experimental import pallas as pl


def kernel(x, edge_index, edge_weights, W_rel, b_rel, W_root):
    raise NotImplementedError("write your pallas kernel here")


# speedup vs baseline: 4.0601x; 4.0601x over previous
import functools

import jax
import jax.numpy as jnp
from jax import lax
from jax.experimental import pallas as pl
from jax.experimental.pallas import tpu as pltpu
from jax.experimental.pallas import tpu_sc as plsc

N_NODES = 34200
D = 128
E_TOTAL = 615600
NE = 342

NC = 2
NS = 16
NW = NC * NS
L = 16

EPT = 19248
EPAD = EPT * NW
SB = 1024
N_SB = 19
SB_LAST = EPT - (N_SB - 1) * SB
BUFW = SB + 16
P = 4
CHUNK = 8552
ACC_ROWS = 8576
ROWS_T = 536
CAP = 21504
B = 128

_mesh = plsc.VectorSubcoreMesh(core_axis_name="c", subcore_axis_name="s")


def _zero16(ref, base):
    ref[pl.ds(base, L)] = jnp.zeros((L,), ref.dtype)


def _sc_body(x_hbm, src_hbm, dst_hbm, wtab_hbm,
             part_out, psrc_hbm, pdst_hbm, pw_hbm,
             wtab_v, sstage, dstage, bsrc, bdst, bw,
             src_v, dst_v, w_v, rows_v, acc, sem):
    c = lax.axis_index("c")
    s = lax.axis_index("s")
    wid = c * NS + s
    ebase = wid * EPT

    pltpu.sync_copy(wtab_hbm, wtab_v)
    rbase = pl.multiple_of(wid * CAP, 8)

    iota = lax.iota(jnp.int32, L)

    offs = [jnp.int32(0)] * P
    for sb in range(N_SB):
        n_sb = SB if sb < N_SB - 1 else SB_LAST
        sb_base = ebase + sb * SB
        pltpu.sync_copy(src_hbm.at[pl.ds(sb_base, n_sb)], sstage.at[pl.ds(0, n_sb)])
        pltpu.sync_copy(dst_hbm.at[pl.ds(sb_base, n_sb)], dstage.at[pl.ds(0, n_sb)])

        def _part(i, loffs):
            s16 = sstage[pl.ds(i * L, L)]
            d16 = dstage[pl.ds(i * L, L)]
            eid = (sb_base + i * L) + iota
            widx = lax.rem(eid, jnp.int32(NE))
            wv = plsc.load_gather(wtab_v, [widx])
            wv = jnp.where(eid < E_TOTAL, wv, 0.0)
            bkt = lax.div(d16, jnp.int32(CHUNK))
            new = []
            for p in range(P):
                m = bkt == p
                lp = p * BUFW + loffs[p]
                plsc.store_compressed(bsrc.at[pl.ds(lp, L)], s16, mask=m)
                plsc.store_compressed(bdst.at[pl.ds(lp, L)], d16, mask=m)
                plsc.store_compressed(bw.at[pl.ds(lp, L)], wv, mask=m)
                new.append(loffs[p] + jnp.sum(m.astype(jnp.int32)))
            return tuple(new)

        loffs = lax.fori_loop(0, n_sb // L, _part, (jnp.int32(0),) * P,
                              unroll=False)

        for p in range(P):
            lp = loffs[p]
            bsrc[pl.ds(p * BUFW + lp, L)] = jnp.zeros((L,), jnp.int32)
            bdst[pl.ds(p * BUFW + lp, L)] = jnp.full((L,), p * CHUNK, jnp.int32)
            bw[pl.ds(p * BUFW + lp, L)] = jnp.zeros((L,), jnp.float32)
            fl = (lp + 7) & ~7
            off = offs[p]
            hoff = pl.multiple_of(rbase + p * NW * CAP + off, 8)
            pltpu.sync_copy(bsrc.at[pl.ds(p * BUFW, BUFW)],
                            psrc_hbm.at[pl.ds(hoff, BUFW)])
            pltpu.sync_copy(bdst.at[pl.ds(p * BUFW, BUFW)],
                            pdst_hbm.at[pl.ds(hoff, BUFW)])
            pltpu.sync_copy(bw.at[pl.ds(p * BUFW, BUFW)],
                            pw_hbm.at[pl.ds(hoff, BUFW)])
            offs[p] = off + fl

    for p in range(P):
        def _zb(i, _):
            for j in range(D // L):
                _zero16(rows_v.at[i], j * L)
            return 0
        lax.fori_loop(0, B, _zb, 0, unroll=False)
        r0 = pl.multiple_of(s * ROWS_T, 8)
        for j in range(4):
            pltpu.sync_copy(rows_v, acc.at[pl.ds(r0 + j * B, B)])
        pltpu.sync_copy(rows_v.at[pl.ds(0, ROWS_T - 4 * B)],
                        acc.at[pl.ds(r0 + 4 * B, ROWS_T - 4 * B)])
        plsc.subcore_barrier()

        cnt = offs[p]
        nblk = lax.div(cnt + (B - 1), jnp.int32(B))

        def _blk(b, _):
            eoff = b * B
            hoff = pl.multiple_of(rbase + p * NW * CAP + eoff, 8)
            pltpu.sync_copy(psrc_hbm.at[pl.ds(hoff, B)], src_v)
            pltpu.sync_copy(pdst_hbm.at[pl.ds(hoff, B)], dst_v)
            pltpu.sync_copy(pw_hbm.at[pl.ds(hoff, B)], w_v)
            for i in range(B // L):
                lane = eoff + i * L + iota
                valid = lane < cnt
                sl = pl.ds(i * L, L)
                src_v[sl] = jnp.where(valid, src_v[sl], 0)
                dst_v[sl] = jnp.where(valid, dst_v[sl] - p * CHUNK, 0)
                w_v[sl] = jnp.where(valid, w_v[sl], 0.0)
            pltpu.async_copy(x_hbm.at[src_v], rows_v, sem).wait()

            def _scale(i, _):
                wsplat = plsc.load_gather(w_v, [jnp.full((L,), i, jnp.int32)])
                for j in range(D // L):
                    sl = pl.ds(j * L, L)
                    rows_v[i, sl] = rows_v[i, sl] * wsplat
                return 0
            lax.fori_loop(0, B, _scale, 0, unroll=False)

            pltpu.sync_copy(rows_v, acc.at[dst_v], add=True)
            return 0

        lax.fori_loop(0, nblk, _blk, 0, unroll=False)
        plsc.subcore_barrier()

        out_r0 = pl.multiple_of(p * CHUNK + s * ROWS_T, 8)
        tail = (CHUNK - 15 * ROWS_T) if p < P - 1 else (N_NODES - (P - 1) * CHUNK - 15 * ROWS_T)

        @pl.when(s < NS - 1)
        def _():
            pltpu.sync_copy(acc.at[pl.ds(r0, ROWS_T)],
                            part_out.at[c, pl.ds(out_r0, ROWS_T)])

        @pl.when(s == NS - 1)
        def _():
            pltpu.sync_copy(acc.at[pl.ds(r0, tail)],
                            part_out.at[c, pl.ds(out_r0, tail)])

        plsc.subcore_barrier()


@jax.jit
def _sc_scatter(x, src, dst, wtab):
    f = pl.kernel(
        _sc_body,
        out_type=(
            jax.ShapeDtypeStruct((NC, N_NODES, D), jnp.float32),
            jax.ShapeDtypeStruct((P * NW * CAP,), jnp.int32),
            jax.ShapeDtypeStruct((P * NW * CAP,), jnp.int32),
            jax.ShapeDtypeStruct((P * NW * CAP,), jnp.float32),
        ),
        mesh=_mesh,
        scratch_types=[
            pltpu.VMEM((1024,), jnp.float32),
            pltpu.VMEM((SB,), jnp.int32),
            pltpu.VMEM((SB,), jnp.int32),
            pltpu.VMEM((P * BUFW,), jnp.int32),
            pltpu.VMEM((P * BUFW,), jnp.int32),
            pltpu.VMEM((P * BUFW,), jnp.float32),
            pltpu.VMEM((B,), jnp.int32),
            pltpu.VMEM((B,), jnp.int32),
            pltpu.VMEM((B,), jnp.float32),
            pltpu.VMEM((B, D), jnp.float32),
            pltpu.VMEM_SHARED((ACC_ROWS, D), jnp.float32),
            pltpu.SemaphoreType.DMA,
        ],
        compiler_params=pltpu.CompilerParams(needs_layout_passes=False),
    )
    return f(x, src, dst, wtab)[0]


def _softplus_body(ew_ref, out_ref):
    out_ref[...] = jnp.logaddexp(ew_ref[...], 0.0)


def _out_body(part_ref, x_ref, wroot_t_ref, wrel_t_ref, b_ref, out_ref):
    agg = part_ref[0] + part_ref[1]
    out_ref[...] = (
        jnp.dot(x_ref[...], wroot_t_ref[...], preferred_element_type=jnp.float32)
        + jnp.dot(agg, wrel_t_ref[...], preferred_element_type=jnp.float32)
        + b_ref[...]
    )


ROW_BLK = 600
N_BLK = N_NODES // ROW_BLK


@jax.jit
def _finish(partial, x, w_root_t, w_rel_t, b2):
    return pl.pallas_call(
        _out_body,
        grid=(N_BLK,),
        in_specs=[
            pl.BlockSpec((NC, ROW_BLK, D), lambda i: (0, i, 0)),
            pl.BlockSpec((ROW_BLK, D), lambda i: (i, 0)),
            pl.BlockSpec((D, D), lambda i: (0, 0)),
            pl.BlockSpec((D, D), lambda i: (0, 0)),
            pl.BlockSpec((1, D), lambda i: (0, 0)),
        ],
        out_specs=pl.BlockSpec((ROW_BLK, D), lambda i: (i, 0)),
        out_shape=jax.ShapeDtypeStruct((N_NODES, D), jnp.float32),
    )(partial, x, w_root_t, w_rel_t, b2)


@jax.jit
def kernel(x, edge_index, edge_weights, W_rel, b_rel, W_root):
    ei = edge_index.astype(jnp.int32)
    pad = jnp.zeros((EPAD - E_TOTAL,), jnp.int32)
    src = jnp.concatenate([ei[0], pad])
    dst = jnp.concatenate([ei[1], pad])

    ew_pad = jnp.zeros((1024,), jnp.float32).at[:NE].set(
        edge_weights.astype(jnp.float32)).reshape(8, 128)
    w_pad = pl.pallas_call(
        _softplus_body,
        out_shape=jax.ShapeDtypeStruct((8, 128), jnp.float32),
    )(ew_pad)
    wtab = w_pad.reshape(1024)

    partial = _sc_scatter(x, src, dst, wtab)
    return _finish(partial, x, W_root.T, W_rel.T, b_rel.reshape(1, D))


# speedup vs baseline: 4.9820x; 1.2271x over previous
import functools

import jax
import jax.numpy as jnp
from jax import lax
from jax.experimental import pallas as pl
from jax.experimental.pallas import tpu as pltpu
from jax.experimental.pallas import tpu_sc as plsc

N_NODES = 34200
D = 128
E_TOTAL = 615600
NE = 342

NC = 2
NS = 16
NW = NC * NS
L = 16

EPT = 19248
EPAD = EPT * NW
SB = 1024
N_SB = 19
SB_LAST = EPT - (N_SB - 1) * SB
BUFW = SB + 16
P = 4
CHUNK = 8552
ACC_ROWS = 8576
ROWS_T = 536
CAP = 21504
B = 128

_mesh = plsc.VectorSubcoreMesh(core_axis_name="c", subcore_axis_name="s")


def _zero16(ref, base):
    ref[pl.ds(base, L)] = jnp.zeros((L,), ref.dtype)


def _sc_body(x_hbm, src_hbm, dst_hbm, wtab_hbm,
             part_out, psrc_hbm, pdst_hbm, pw_hbm,
             wtab_v, sstage, dstage, bsrc, bdst, bw,
             src_v0, dst_v0, w_v0, rows_v0,
             src_v1, dst_v1, w_v1, rows_v1, acc, sem0, sem1, fsem):
    c = lax.axis_index("c")
    s = lax.axis_index("s")
    wid = c * NS + s
    ebase = wid * EPT

    pltpu.sync_copy(wtab_hbm, wtab_v)
    rbase = pl.multiple_of(wid * CAP, 8)

    iota = lax.iota(jnp.int32, L)

    offs = [jnp.int32(0)] * P
    flush_descs = []
    for sb in range(N_SB):
        n_sb = SB if sb < N_SB - 1 else SB_LAST
        sb_base = ebase + sb * SB
        pltpu.sync_copy(src_hbm.at[pl.ds(sb_base, n_sb)], sstage.at[pl.ds(0, n_sb)])
        pltpu.sync_copy(dst_hbm.at[pl.ds(sb_base, n_sb)], dstage.at[pl.ds(0, n_sb)])
        for d in flush_descs:
            d.wait()
        flush_descs = []

        def _part(i, loffs):
            s16 = sstage[pl.ds(i * L, L)]
            d16 = dstage[pl.ds(i * L, L)]
            eid = (sb_base + i * L) + iota
            widx = lax.rem(eid, jnp.int32(NE))
            wv = plsc.load_gather(wtab_v, [widx])
            wv = jnp.where(eid < E_TOTAL, wv, 0.0)
            bkt = lax.div(d16, jnp.int32(CHUNK))
            new = []
            for p in range(P):
                m = bkt == p
                lp = p * BUFW + loffs[p]
                plsc.store_compressed(bsrc.at[pl.ds(lp, L)], s16, mask=m)
                plsc.store_compressed(bdst.at[pl.ds(lp, L)], d16, mask=m)
                plsc.store_compressed(bw.at[pl.ds(lp, L)], wv, mask=m)
                new.append(loffs[p] + jnp.sum(m.astype(jnp.int32)))
            return tuple(new)

        loffs = lax.fori_loop(0, n_sb // L, _part, (jnp.int32(0),) * P,
                              unroll=False)

        for p in range(P):
            lp = loffs[p]
            bsrc[pl.ds(p * BUFW + lp, L)] = jnp.zeros((L,), jnp.int32)
            bdst[pl.ds(p * BUFW + lp, L)] = jnp.full((L,), p * CHUNK, jnp.int32)
            bw[pl.ds(p * BUFW + lp, L)] = jnp.zeros((L,), jnp.float32)
            fl = (lp + 7) & ~7
            off = offs[p]
            hoff = pl.multiple_of(rbase + p * NW * CAP + off, 8)
            flush_descs.append(pltpu.async_copy(
                bsrc.at[pl.ds(p * BUFW, BUFW)],
                psrc_hbm.at[pl.ds(hoff, BUFW)], fsem))
            flush_descs.append(pltpu.async_copy(
                bdst.at[pl.ds(p * BUFW, BUFW)],
                pdst_hbm.at[pl.ds(hoff, BUFW)], fsem))
            flush_descs.append(pltpu.async_copy(
                bw.at[pl.ds(p * BUFW, BUFW)],
                pw_hbm.at[pl.ds(hoff, BUFW)], fsem))
            offs[p] = off + fl
    for d in flush_descs:
        d.wait()

    for p in range(P):
        def _zb(i, _):
            for j in range(D // L):
                _zero16(rows_v0.at[i], j * L)
            return 0
        lax.fori_loop(0, B, _zb, 0, unroll=False)
        r0 = pl.multiple_of(s * ROWS_T, 8)
        for j in range(4):
            pltpu.sync_copy(rows_v0, acc.at[pl.ds(r0 + j * B, B)])
        pltpu.sync_copy(rows_v0.at[pl.ds(0, ROWS_T - 4 * B)],
                        acc.at[pl.ds(r0 + 4 * B, ROWS_T - 4 * B)])
        plsc.subcore_barrier()

        cnt = offs[p]
        nblk = lax.div(cnt + (B - 1), jnp.int32(B))
        bufs = ((src_v0, dst_v0, w_v0, rows_v0, sem0),
                (src_v1, dst_v1, w_v1, rows_v1, sem1))

        def _fire(b, k):
            src_v, dst_v, w_v, rows_v, sem = bufs[k]
            eoff = b * B
            hoff = pl.multiple_of(rbase + p * NW * CAP + eoff, 8)
            pltpu.sync_copy(psrc_hbm.at[pl.ds(hoff, B)], src_v)
            pltpu.sync_copy(pdst_hbm.at[pl.ds(hoff, B)], dst_v)
            pltpu.sync_copy(pw_hbm.at[pl.ds(hoff, B)], w_v)
            for i in range(B // L):
                lane = eoff + i * L + iota
                valid = lane < cnt
                sl = pl.ds(i * L, L)
                src_v[sl] = jnp.where(valid, src_v[sl], 0)
                dst_v[sl] = jnp.where(valid, dst_v[sl] - p * CHUNK, 0)
                w_v[sl] = jnp.where(valid, w_v[sl], 0.0)
            pltpu.async_copy(x_hbm.at[src_v], rows_v, sem)

        def _process(k):
            src_v, dst_v, w_v, rows_v, sem = bufs[k]
            pltpu.make_async_copy(x_hbm.at[src_v], rows_v, sem).wait()

            def _scale(i, _):
                wsplat = plsc.load_gather(w_v, [jnp.full((L,), i, jnp.int32)])
                for j in range(D // L):
                    sl = pl.ds(j * L, L)
                    rows_v[i, sl] = rows_v[i, sl] * wsplat
                return 0
            lax.fori_loop(0, B, _scale, 0, unroll=2)

            pltpu.sync_copy(rows_v, acc.at[dst_v], add=True)

        @pl.when(nblk > 0)
        def _():
            _fire(jnp.int32(0), 0)

        def _super(g, _):
            for k in range(2):
                b = g * 2 + k

                @pl.when(b + 1 < nblk)
                def _():
                    _fire(b + 1, 1 - k)

                @pl.when(b < nblk)
                def _():
                    _process(k)
            return 0

        lax.fori_loop(0, lax.div(nblk + 1, jnp.int32(2)), _super, 0,
                      unroll=False)
        plsc.subcore_barrier()

        out_r0 = pl.multiple_of(p * CHUNK + s * ROWS_T, 8)
        tail = (CHUNK - 15 * ROWS_T) if p < P - 1 else (N_NODES - (P - 1) * CHUNK - 15 * ROWS_T)

        @pl.when(s < NS - 1)
        def _():
            pltpu.sync_copy(acc.at[pl.ds(r0, ROWS_T)],
                            part_out.at[c, pl.ds(out_r0, ROWS_T)])

        @pl.when(s == NS - 1)
        def _():
            pltpu.sync_copy(acc.at[pl.ds(r0, tail)],
                            part_out.at[c, pl.ds(out_r0, tail)])

        plsc.subcore_barrier()


@jax.jit
def _sc_scatter(x, src, dst, wtab):
    f = pl.kernel(
        _sc_body,
        out_type=(
            jax.ShapeDtypeStruct((NC, N_NODES, D), jnp.float32),
            jax.ShapeDtypeStruct((P * NW * CAP,), jnp.int32),
            jax.ShapeDtypeStruct((P * NW * CAP,), jnp.int32),
            jax.ShapeDtypeStruct((P * NW * CAP,), jnp.float32),
        ),
        mesh=_mesh,
        scratch_types=[
            pltpu.VMEM((1024,), jnp.float32),
            pltpu.VMEM((SB,), jnp.int32),
            pltpu.VMEM((SB,), jnp.int32),
            pltpu.VMEM((P * BUFW,), jnp.int32),
            pltpu.VMEM((P * BUFW,), jnp.int32),
            pltpu.VMEM((P * BUFW,), jnp.float32),
            pltpu.VMEM((B,), jnp.int32),
            pltpu.VMEM((B,), jnp.int32),
            pltpu.VMEM((B,), jnp.float32),
            pltpu.VMEM((B, D), jnp.float32),
            pltpu.VMEM((B,), jnp.int32),
            pltpu.VMEM((B,), jnp.int32),
            pltpu.VMEM((B,), jnp.float32),
            pltpu.VMEM((B, D), jnp.float32),
            pltpu.VMEM_SHARED((ACC_ROWS, D), jnp.float32),
            pltpu.SemaphoreType.DMA,
            pltpu.SemaphoreType.DMA,
            pltpu.SemaphoreType.DMA,
        ],
        compiler_params=pltpu.CompilerParams(needs_layout_passes=False),
    )
    return f(x, src, dst, wtab)[0]


def _softplus_body(ew_ref, out_ref):
    out_ref[...] = jnp.logaddexp(ew_ref[...], 0.0)


def _out_body(part_ref, x_ref, wroot_t_ref, wrel_t_ref, b_ref, out_ref):
    agg = part_ref[0] + part_ref[1]
    out_ref[...] = (
        jnp.dot(x_ref[...], wroot_t_ref[...], preferred_element_type=jnp.float32)
        + jnp.dot(agg, wrel_t_ref[...], preferred_element_type=jnp.float32)
        + b_ref[...]
    )


ROW_BLK = 600
N_BLK = N_NODES // ROW_BLK


@jax.jit
def _finish(partial, x, w_root_t, w_rel_t, b2):
    return pl.pallas_call(
        _out_body,
        grid=(N_BLK,),
        in_specs=[
            pl.BlockSpec((NC, ROW_BLK, D), lambda i: (0, i, 0)),
            pl.BlockSpec((ROW_BLK, D), lambda i: (i, 0)),
            pl.BlockSpec((D, D), lambda i: (0, 0)),
            pl.BlockSpec((D, D), lambda i: (0, 0)),
            pl.BlockSpec((1, D), lambda i: (0, 0)),
        ],
        out_specs=pl.BlockSpec((ROW_BLK, D), lambda i: (i, 0)),
        out_shape=jax.ShapeDtypeStruct((N_NODES, D), jnp.float32),
    )(partial, x, w_root_t, w_rel_t, b2)


@jax.jit
def kernel(x, edge_index, edge_weights, W_rel, b_rel, W_root):
    ei = edge_index.astype(jnp.int32)
    pad = jnp.zeros((EPAD - E_TOTAL,), jnp.int32)
    src = jnp.concatenate([ei[0], pad])
    dst = jnp.concatenate([ei[1], pad])

    ew_pad = jnp.zeros((1024,), jnp.float32).at[:NE].set(
        edge_weights.astype(jnp.float32)).reshape(8, 128)
    w_pad = pl.pallas_call(
        _softplus_body,
        out_shape=jax.ShapeDtypeStruct((8, 128), jnp.float32),
    )(ew_pad)
    wtab = w_pad.reshape(1024)

    partial = _sc_scatter(x, src, dst, wtab)
    return _finish(partial, x, W_root.T, W_rel.T, b_rel.reshape(1, D))


# speedup vs baseline: 5.2213x; 1.0480x over previous
import functools

import jax
import jax.numpy as jnp
from jax import lax
from jax.experimental import pallas as pl
from jax.experimental.pallas import tpu as pltpu
from jax.experimental.pallas import tpu_sc as plsc

N_NODES = 34200
D = 128
E_TOTAL = 615600
NE = 342

NC = 2
NS = 16
NW = NC * NS
L = 16

EPT = 19248
EPAD = EPT * NW
SB = 1024
N_SB = 19
SB_LAST = EPT - (N_SB - 1) * SB
BUFW = SB + 16
P = 4
CHUNK = 8552
ACC_ROWS = 8576
ROWS_T = 536
CAP = 21504
B = 128

_mesh = plsc.VectorSubcoreMesh(core_axis_name="c", subcore_axis_name="s")


def _zero16(ref, base):
    ref[pl.ds(base, L)] = jnp.zeros((L,), ref.dtype)


def _sc_body(x_hbm, src_hbm, dst_hbm, wtab_hbm,
             part_out, psrc_hbm, pw_hbm,
             wtab_v, sstage, dstage, bsrc, bw,
             src_v0, dst_v0, w_v0, rows_v0,
             src_v1, dst_v1, w_v1, rows_v1, acc,
             sem0, sem1, ssem0, ssem1, fsem):
    c = lax.axis_index("c")
    s = lax.axis_index("s")
    wid = c * NS + s
    ebase = wid * EPT

    pltpu.sync_copy(wtab_hbm, wtab_v)
    rbase = pl.multiple_of(wid * CAP, 8)

    iota = lax.iota(jnp.int32, L)

    offs = [jnp.int32(0)] * P
    flush_descs = []
    for sb in range(N_SB):
        n_sb = SB if sb < N_SB - 1 else SB_LAST
        sb_base = ebase + sb * SB
        pltpu.sync_copy(src_hbm.at[pl.ds(sb_base, n_sb)], sstage.at[pl.ds(0, n_sb)])
        pltpu.sync_copy(dst_hbm.at[pl.ds(sb_base, n_sb)], dstage.at[pl.ds(0, n_sb)])
        for d in flush_descs:
            d.wait()
        flush_descs = []

        def _part(i, loffs):
            s16 = sstage[pl.ds(i * L, L)]
            d16 = dstage[pl.ds(i * L, L)]
            eid = (sb_base + i * L) + iota
            widx = lax.rem(eid, jnp.int32(NE))
            wv = plsc.load_gather(wtab_v, [widx])
            wv = jnp.where(eid < E_TOTAL, wv, 0.0)
            bkt = lax.div(d16, jnp.int32(CHUNK))
            sd16 = s16 | lax.shift_left(d16, 16)
            new = []
            for p in range(P):
                m = bkt == p
                lp = p * BUFW + loffs[p]
                plsc.store_compressed(bsrc.at[pl.ds(lp, L)], sd16, mask=m)
                plsc.store_compressed(bw.at[pl.ds(lp, L)], wv, mask=m)
                new.append(loffs[p] + jnp.sum(m.astype(jnp.int32)))
            return tuple(new)

        loffs = lax.fori_loop(0, n_sb // L, _part, (jnp.int32(0),) * P,
                              unroll=False)

        for p in range(P):
            lp = loffs[p]
            bsrc[pl.ds(p * BUFW + lp, L)] = jnp.full(
                (L,), p * CHUNK << 16, jnp.int32)
            bw[pl.ds(p * BUFW + lp, L)] = jnp.zeros((L,), jnp.float32)
            fl = (lp + 7) & ~7
            off = offs[p]
            hoff = pl.multiple_of(rbase + p * NW * CAP + off, 8)
            flush_descs.append(pltpu.async_copy(
                bsrc.at[pl.ds(p * BUFW, BUFW)],
                psrc_hbm.at[pl.ds(hoff, BUFW)], fsem))
            flush_descs.append(pltpu.async_copy(
                bw.at[pl.ds(p * BUFW, BUFW)],
                pw_hbm.at[pl.ds(hoff, BUFW)], fsem))
            offs[p] = off + fl
    for d in flush_descs:
        d.wait()

    for p in range(P):
        def _zb(i, _):
            for j in range(D // L):
                _zero16(rows_v0.at[i], j * L)
            return 0
        lax.fori_loop(0, B, _zb, 0, unroll=False)
        r0 = pl.multiple_of(s * ROWS_T, 8)
        for j in range(4):
            pltpu.sync_copy(rows_v0, acc.at[pl.ds(r0 + j * B, B)])
        pltpu.sync_copy(rows_v0.at[pl.ds(0, ROWS_T - 4 * B)],
                        acc.at[pl.ds(r0 + 4 * B, ROWS_T - 4 * B)])
        plsc.subcore_barrier()

        cnt = offs[p]
        nblk = lax.div(cnt + (B - 1), jnp.int32(B))
        bufs = ((src_v0, dst_v0, w_v0, rows_v0, sem0, ssem0),
                (src_v1, dst_v1, w_v1, rows_v1, sem1, ssem1))

        def _fire(b, k):
            src_v, dst_v, w_v, rows_v, sem, ssem = bufs[k]
            eoff = b * B
            hoff = pl.multiple_of(rbase + p * NW * CAP + eoff, 8)
            pltpu.sync_copy(psrc_hbm.at[pl.ds(hoff, B)], src_v)
            pltpu.sync_copy(pw_hbm.at[pl.ds(hoff, B)], w_v)
            @pl.when(b >= 2)
            def _():
                pltpu.make_async_copy(rows_v, acc.at[dst_v], ssem).wait()
            for i in range(B // L):
                lane = eoff + i * L + iota
                valid = lane < cnt
                sl = pl.ds(i * L, L)
                sd = src_v[sl]
                src_v[sl] = jnp.where(valid, sd & 0xFFFF, 0)
                dst_v[sl] = jnp.where(
                    valid, lax.shift_right_logical(sd, 16) - p * CHUNK, 0)
                w_v[sl] = jnp.where(valid, w_v[sl], 0.0)
            pltpu.async_copy(x_hbm.at[src_v], rows_v, sem)

        def _process(k):
            src_v, dst_v, w_v, rows_v, sem, ssem = bufs[k]
            pltpu.make_async_copy(x_hbm.at[src_v], rows_v, sem).wait()

            def _scale(i, _):
                wsplat = plsc.load_gather(w_v, [jnp.full((L,), i, jnp.int32)])
                for j in range(D // L):
                    sl = pl.ds(j * L, L)
                    rows_v[i, sl] = rows_v[i, sl] * wsplat
                return 0
            lax.fori_loop(0, B, _scale, 0, unroll=2)

            pltpu.async_copy(rows_v, acc.at[dst_v], ssem, add=True)

        @pl.when(nblk > 0)
        def _():
            _fire(jnp.int32(0), 0)

        def _super(g, _):
            for k in range(2):
                b = g * 2 + k

                @pl.when(b + 1 < nblk)
                def _():
                    _fire(b + 1, 1 - k)

                @pl.when(b < nblk)
                def _():
                    _process(k)
            return 0

        lax.fori_loop(0, lax.div(nblk + 1, jnp.int32(2)), _super, 0,
                      unroll=False)

        @pl.when(nblk >= 2)
        def _():
            for k in range(2):
                src_v, dst_v, w_v, rows_v, sem, ssem = bufs[k]
                pltpu.make_async_copy(rows_v, acc.at[dst_v], ssem).wait()

        @pl.when(nblk == 1)
        def _():
            src_v, dst_v, w_v, rows_v, sem, ssem = bufs[0]
            pltpu.make_async_copy(rows_v, acc.at[dst_v], ssem).wait()

        plsc.subcore_barrier()

        out_r0 = pl.multiple_of(p * CHUNK + s * ROWS_T, 8)
        tail = (CHUNK - 15 * ROWS_T) if p < P - 1 else (N_NODES - (P - 1) * CHUNK - 15 * ROWS_T)

        @pl.when(s < NS - 1)
        def _():
            pltpu.sync_copy(acc.at[pl.ds(r0, ROWS_T)],
                            part_out.at[c, pl.ds(out_r0, ROWS_T)])

        @pl.when(s == NS - 1)
        def _():
            pltpu.sync_copy(acc.at[pl.ds(r0, tail)],
                            part_out.at[c, pl.ds(out_r0, tail)])

        plsc.subcore_barrier()


@jax.jit
def _sc_scatter(x, src, dst, wtab):
    f = pl.kernel(
        _sc_body,
        out_type=(
            jax.ShapeDtypeStruct((NC, N_NODES, D), jnp.float32),
            jax.ShapeDtypeStruct((P * NW * CAP,), jnp.int32),
            jax.ShapeDtypeStruct((P * NW * CAP,), jnp.float32),
        ),
        mesh=_mesh,
        scratch_types=[
            pltpu.VMEM((1024,), jnp.float32),
            pltpu.VMEM((SB,), jnp.int32),
            pltpu.VMEM((SB,), jnp.int32),
            pltpu.VMEM((P * BUFW,), jnp.int32),
            pltpu.VMEM((P * BUFW,), jnp.float32),
            pltpu.VMEM((B,), jnp.int32),
            pltpu.VMEM((B,), jnp.int32),
            pltpu.VMEM((B,), jnp.float32),
            pltpu.VMEM((B, D), jnp.float32),
            pltpu.VMEM((B,), jnp.int32),
            pltpu.VMEM((B,), jnp.int32),
            pltpu.VMEM((B,), jnp.float32),
            pltpu.VMEM((B, D), jnp.float32),
            pltpu.VMEM_SHARED((ACC_ROWS, D), jnp.float32),
            pltpu.SemaphoreType.DMA,
            pltpu.SemaphoreType.DMA,
            pltpu.SemaphoreType.DMA,
            pltpu.SemaphoreType.DMA,
            pltpu.SemaphoreType.DMA,
        ],
        compiler_params=pltpu.CompilerParams(needs_layout_passes=False),
    )
    return f(x, src, dst, wtab)[0]


def _softplus_body(ew_ref, out_ref):
    out_ref[...] = jnp.logaddexp(ew_ref[...], 0.0)


def _out_body(part_ref, x_ref, wroot_t_ref, wrel_t_ref, b_ref, out_ref):
    agg = part_ref[0] + part_ref[1]
    out_ref[...] = (
        jnp.dot(x_ref[...], wroot_t_ref[...], preferred_element_type=jnp.float32)
        + jnp.dot(agg, wrel_t_ref[...], preferred_element_type=jnp.float32)
        + b_ref[...]
    )


ROW_BLK = 600
N_BLK = N_NODES // ROW_BLK


@jax.jit
def _finish(partial, x, w_root_t, w_rel_t, b2):
    return pl.pallas_call(
        _out_body,
        grid=(N_BLK,),
        in_specs=[
            pl.BlockSpec((NC, ROW_BLK, D), lambda i: (0, i, 0)),
            pl.BlockSpec((ROW_BLK, D), lambda i: (i, 0)),
            pl.BlockSpec((D, D), lambda i: (0, 0)),
            pl.BlockSpec((D, D), lambda i: (0, 0)),
            pl.BlockSpec((1, D), lambda i: (0, 0)),
        ],
        out_specs=pl.BlockSpec((ROW_BLK, D), lambda i: (i, 0)),
        out_shape=jax.ShapeDtypeStruct((N_NODES, D), jnp.float32),
    )(partial, x, w_root_t, w_rel_t, b2)


@jax.jit
def kernel(x, edge_index, edge_weights, W_rel, b_rel, W_root):
    ei = edge_index.astype(jnp.int32)
    pad = jnp.zeros((EPAD - E_TOTAL,), jnp.int32)
    src = jnp.concatenate([ei[0], pad])
    dst = jnp.concatenate([ei[1], pad])

    ew_pad = jnp.zeros((1024,), jnp.float32).at[:NE].set(
        edge_weights.astype(jnp.float32)).reshape(8, 128)
    w_pad = pl.pallas_call(
        _softplus_body,
        out_shape=jax.ShapeDtypeStruct((8, 128), jnp.float32),
    )(ew_pad)
    wtab = w_pad.reshape(1024)

    partial = _sc_scatter(x, src, dst, wtab)
    return _finish(partial, x, W_root.T, W_rel.T, b_rel.reshape(1, D))


# speedup vs baseline: 5.2326x; 1.0022x over previous
import functools

import jax
import jax.numpy as jnp
from jax import lax
from jax.experimental import pallas as pl
from jax.experimental.pallas import tpu as pltpu
from jax.experimental.pallas import tpu_sc as plsc

N_NODES = 34200
D = 128
E_TOTAL = 615600
NE = 342

NC = 2
NS = 16
NW = NC * NS
L = 16

EPT = 19248
EPAD = EPT * NW
SB = 1024
N_SB = 19
SB_LAST = EPT - (N_SB - 1) * SB
BUFW = SB + 16
P = 4
CHUNK = 8552
ACC_ROWS = 8576
ROWS_T = 536
CAP = 21504
B = 128

_mesh = plsc.VectorSubcoreMesh(core_axis_name="c", subcore_axis_name="s")


def _zero16(ref, base):
    ref[pl.ds(base, L)] = jnp.zeros((L,), ref.dtype)


def _sc_body(x_hbm, src_hbm, dst_hbm, wtab_hbm,
             part_out, psrc_hbm, pw_hbm,
             wtab_v, sstage, dstage, bsrc, bw,
             src_v0, dst_v0, w_v0, rows_v0,
             src_v1, dst_v1, w_v1, rows_v1, acc,
             sem0, sem1, ssem0, ssem1, isem0, isem1, fsem):
    c = lax.axis_index("c")
    s = lax.axis_index("s")
    wid = c * NS + s
    ebase = wid * EPT

    pltpu.sync_copy(wtab_hbm, wtab_v)
    rbase = pl.multiple_of(wid * CAP, 8)

    iota = lax.iota(jnp.int32, L)

    offs = [jnp.int32(0)] * P
    flush_descs = []
    for sb in range(N_SB):
        n_sb = SB if sb < N_SB - 1 else SB_LAST
        sb_base = ebase + sb * SB
        pltpu.sync_copy(src_hbm.at[pl.ds(sb_base, n_sb)], sstage.at[pl.ds(0, n_sb)])
        pltpu.sync_copy(dst_hbm.at[pl.ds(sb_base, n_sb)], dstage.at[pl.ds(0, n_sb)])
        for d in flush_descs:
            d.wait()
        flush_descs = []

        def _part(i, loffs):
            s16 = sstage[pl.ds(i * L, L)]
            d16 = dstage[pl.ds(i * L, L)]
            eid = (sb_base + i * L) + iota
            widx = lax.rem(eid, jnp.int32(NE))
            wv = plsc.load_gather(wtab_v, [widx])
            wv = jnp.where(eid < E_TOTAL, wv, 0.0)
            bkt = lax.div(d16, jnp.int32(CHUNK))
            sd16 = s16 | lax.shift_left(d16, 16)
            new = []
            for p in range(P):
                m = bkt == p
                lp = p * BUFW + loffs[p]
                plsc.store_compressed(bsrc.at[pl.ds(lp, L)], sd16, mask=m)
                plsc.store_compressed(bw.at[pl.ds(lp, L)], wv, mask=m)
                new.append(loffs[p] + jnp.sum(m.astype(jnp.int32)))
            return tuple(new)

        loffs = lax.fori_loop(0, n_sb // L, _part, (jnp.int32(0),) * P,
                              unroll=False)

        for p in range(P):
            lp = loffs[p]
            bsrc[pl.ds(p * BUFW + lp, L)] = jnp.full(
                (L,), p * CHUNK << 16, jnp.int32)
            bw[pl.ds(p * BUFW + lp, L)] = jnp.zeros((L,), jnp.float32)
            fl = (lp + 7) & ~7
            off = offs[p]
            hoff = pl.multiple_of(rbase + p * NW * CAP + off, 8)
            flush_descs.append(pltpu.async_copy(
                bsrc.at[pl.ds(p * BUFW, BUFW)],
                psrc_hbm.at[pl.ds(hoff, BUFW)], fsem))
            flush_descs.append(pltpu.async_copy(
                bw.at[pl.ds(p * BUFW, BUFW)],
                pw_hbm.at[pl.ds(hoff, BUFW)], fsem))
            offs[p] = off + fl
    for d in flush_descs:
        d.wait()

    for p in range(P):
        def _zb(i, _):
            for j in range(D // L):
                _zero16(rows_v0.at[i], j * L)
            return 0
        lax.fori_loop(0, B, _zb, 0, unroll=False)
        r0 = pl.multiple_of(s * ROWS_T, 8)
        for j in range(4):
            pltpu.sync_copy(rows_v0, acc.at[pl.ds(r0 + j * B, B)])
        pltpu.sync_copy(rows_v0.at[pl.ds(0, ROWS_T - 4 * B)],
                        acc.at[pl.ds(r0 + 4 * B, ROWS_T - 4 * B)])
        plsc.subcore_barrier()

        cnt = offs[p]
        nblk = lax.div(cnt + (B - 1), jnp.int32(B))
        bufs = ((src_v0, dst_v0, w_v0, rows_v0, sem0, ssem0, isem0),
                (src_v1, dst_v1, w_v1, rows_v1, sem1, ssem1, isem1))

        def _idx_load(b, k, sync=False):
            src_v, dst_v, w_v, rows_v, sem, ssem, isem = bufs[k]
            eoff = b * B
            hoff = pl.multiple_of(rbase + p * NW * CAP + eoff, 8)
            if sync:
                pltpu.sync_copy(psrc_hbm.at[pl.ds(hoff, B)], src_v)
                pltpu.sync_copy(pw_hbm.at[pl.ds(hoff, B)], w_v)
            else:
                pltpu.async_copy(psrc_hbm.at[pl.ds(hoff, B)], src_v, isem)
                pltpu.async_copy(pw_hbm.at[pl.ds(hoff, B)], w_v, isem)

        def _idx_wait(b, k):
            src_v, dst_v, w_v, rows_v, sem, ssem, isem = bufs[k]
            eoff = b * B
            hoff = pl.multiple_of(rbase + p * NW * CAP + eoff, 8)
            pltpu.make_async_copy(psrc_hbm.at[pl.ds(hoff, B)], src_v, isem).wait()
            pltpu.make_async_copy(pw_hbm.at[pl.ds(hoff, B)], w_v, isem).wait()

        def _sanitize_fire(b, k):
            src_v, dst_v, w_v, rows_v, sem, ssem, isem = bufs[k]
            eoff = b * B
            @pl.when(b >= 2)
            def _():
                pltpu.make_async_copy(rows_v, acc.at[dst_v], ssem).wait()
            for i in range(B // L):
                lane = eoff + i * L + iota
                valid = lane < cnt
                sl = pl.ds(i * L, L)
                sd = src_v[sl]
                src_v[sl] = jnp.where(valid, sd & 0xFFFF, 0)
                dst_v[sl] = jnp.where(
                    valid, lax.shift_right_logical(sd, 16) - p * CHUNK, 0)
                w_v[sl] = jnp.where(valid, w_v[sl], 0.0)
            pltpu.async_copy(x_hbm.at[src_v], rows_v, sem)

        def _process(b, k):
            src_v, dst_v, w_v, rows_v, sem, ssem, isem = bufs[k]
            pltpu.make_async_copy(x_hbm.at[src_v], rows_v, sem).wait()

            @pl.when(b + 2 < nblk)
            def _():
                _idx_load(b + 2, k)

            def _scale(i, _):
                wsplat = plsc.load_gather(w_v, [jnp.full((L,), i, jnp.int32)])
                for j in range(D // L):
                    sl = pl.ds(j * L, L)
                    rows_v[i, sl] = rows_v[i, sl] * wsplat
                return 0
            lax.fori_loop(0, B, _scale, 0, unroll=4)

            pltpu.async_copy(rows_v, acc.at[dst_v], ssem, add=True)

        @pl.when(nblk > 0)
        def _():
            _idx_load(jnp.int32(0), 0, sync=True)
            _sanitize_fire(jnp.int32(0), 0)

        @pl.when(nblk > 1)
        def _():
            _idx_load(jnp.int32(1), 1)

        def _super(g, _):
            for k in range(2):
                b = g * 2 + k

                @pl.when(b + 1 < nblk)
                def _():
                    _idx_wait(b + 1, 1 - k)
                    _sanitize_fire(b + 1, 1 - k)

                @pl.when(b < nblk)
                def _():
                    _process(b, k)
            return 0

        lax.fori_loop(0, lax.div(nblk + 1, jnp.int32(2)), _super, 0,
                      unroll=False)

        @pl.when(nblk >= 2)
        def _():
            for k in range(2):
                src_v, dst_v, w_v, rows_v, sem, ssem, isem = bufs[k]
                pltpu.make_async_copy(rows_v, acc.at[dst_v], ssem).wait()

        @pl.when(nblk == 1)
        def _():
            src_v, dst_v, w_v, rows_v, sem, ssem, isem = bufs[0]
            pltpu.make_async_copy(rows_v, acc.at[dst_v], ssem).wait()

        plsc.subcore_barrier()

        out_r0 = pl.multiple_of(p * CHUNK + s * ROWS_T, 8)
        tail = (CHUNK - 15 * ROWS_T) if p < P - 1 else (N_NODES - (P - 1) * CHUNK - 15 * ROWS_T)

        @pl.when(s < NS - 1)
        def _():
            pltpu.sync_copy(acc.at[pl.ds(r0, ROWS_T)],
                            part_out.at[c, pl.ds(out_r0, ROWS_T)])

        @pl.when(s == NS - 1)
        def _():
            pltpu.sync_copy(acc.at[pl.ds(r0, tail)],
                            part_out.at[c, pl.ds(out_r0, tail)])

        plsc.subcore_barrier()


@jax.jit
def _sc_scatter(x, src, dst, wtab):
    f = pl.kernel(
        _sc_body,
        out_type=(
            jax.ShapeDtypeStruct((NC, N_NODES, D), jnp.float32),
            jax.ShapeDtypeStruct((P * NW * CAP,), jnp.int32),
            jax.ShapeDtypeStruct((P * NW * CAP,), jnp.float32),
        ),
        mesh=_mesh,
        scratch_types=[
            pltpu.VMEM((1024,), jnp.float32),
            pltpu.VMEM((SB,), jnp.int32),
            pltpu.VMEM((SB,), jnp.int32),
            pltpu.VMEM((P * BUFW,), jnp.int32),
            pltpu.VMEM((P * BUFW,), jnp.float32),
            pltpu.VMEM((B,), jnp.int32),
            pltpu.VMEM((B,), jnp.int32),
            pltpu.VMEM((B,), jnp.float32),
            pltpu.VMEM((B, D), jnp.float32),
            pltpu.VMEM((B,), jnp.int32),
            pltpu.VMEM((B,), jnp.int32),
            pltpu.VMEM((B,), jnp.float32),
            pltpu.VMEM((B, D), jnp.float32),
            pltpu.VMEM_SHARED((ACC_ROWS, D), jnp.float32),
            pltpu.SemaphoreType.DMA,
            pltpu.SemaphoreType.DMA,
            pltpu.SemaphoreType.DMA,
            pltpu.SemaphoreType.DMA,
            pltpu.SemaphoreType.DMA,
            pltpu.SemaphoreType.DMA,
            pltpu.SemaphoreType.DMA,
        ],
        compiler_params=pltpu.CompilerParams(needs_layout_passes=False),
    )
    return f(x, src, dst, wtab)[0]


def _softplus_body(ew_ref, out_ref):
    out_ref[...] = jnp.logaddexp(ew_ref[...], 0.0)


def _out_body(part_ref, x_ref, wroot_t_ref, wrel_t_ref, b_ref, out_ref):
    agg = part_ref[0] + part_ref[1]
    out_ref[...] = (
        jnp.dot(x_ref[...], wroot_t_ref[...], preferred_element_type=jnp.float32)
        + jnp.dot(agg, wrel_t_ref[...], preferred_element_type=jnp.float32)
        + b_ref[...]
    )


ROW_BLK = 600
N_BLK = N_NODES // ROW_BLK


@jax.jit
def _finish(partial, x, w_root_t, w_rel_t, b2):
    return pl.pallas_call(
        _out_body,
        grid=(N_BLK,),
        in_specs=[
            pl.BlockSpec((NC, ROW_BLK, D), lambda i: (0, i, 0)),
            pl.BlockSpec((ROW_BLK, D), lambda i: (i, 0)),
            pl.BlockSpec((D, D), lambda i: (0, 0)),
            pl.BlockSpec((D, D), lambda i: (0, 0)),
            pl.BlockSpec((1, D), lambda i: (0, 0)),
        ],
        out_specs=pl.BlockSpec((ROW_BLK, D), lambda i: (i, 0)),
        out_shape=jax.ShapeDtypeStruct((N_NODES, D), jnp.float32),
    )(partial, x, w_root_t, w_rel_t, b2)


@jax.jit
def kernel(x, edge_index, edge_weights, W_rel, b_rel, W_root):
    ei = edge_index.astype(jnp.int32)
    pad = jnp.zeros((EPAD - E_TOTAL,), jnp.int32)
    src = jnp.concatenate([ei[0], pad])
    dst = jnp.concatenate([ei[1], pad])

    ew_pad = jnp.zeros((1024,), jnp.float32).at[:NE].set(
        edge_weights.astype(jnp.float32)).reshape(8, 128)
    w_pad = pl.pallas_call(
        _softplus_body,
        out_shape=jax.ShapeDtypeStruct((8, 128), jnp.float32),
    )(ew_pad)
    wtab = w_pad.reshape(1024)

    partial = _sc_scatter(x, src, dst, wtab)
    return _finish(partial, x, W_root.T, W_rel.T, b_rel.reshape(1, D))
